# baseline (speedup 1.0000x reference)
import functools

import jax
import jax.numpy as jnp
from jax import lax
from jax.experimental import pallas as pl
from jax.experimental.pallas import tpu as pltpu
from jax.experimental.pallas import tpu_sc as plsc

_HIDDEN = 1024
_B = 4 * 4096
_NC, _NS = 2, 16
_NW = _NC * _NS
_BPW = _B // _NW
_CH = 32
_NCHUNK = _BPW // _CH

_mesh = plsc.VectorSubcoreMesh(core_axis_name="c", subcore_axis_name="s")


@functools.partial(
    pl.kernel,
    mesh=_mesh,
    out_type=jax.ShapeDtypeStruct((_B, _HIDDEN), jnp.float32),
    scratch_types=[
        pltpu.VMEM((_NCHUNK, _CH), jnp.int32),
        pltpu.VMEM((_CH, _HIDDEN), jnp.float32),
        pltpu.VMEM((_CH, _HIDDEN), jnp.float32),
        pltpu.SemaphoreType.DMA,
        pltpu.SemaphoreType.DMA,
        pltpu.SemaphoreType.DMA,
        pltpu.SemaphoreType.DMA,
    ],
)
def _sc_gather(idx_hbm, table_hbm, out_hbm, idx_v, buf0, buf1, g0, g1, s0, s1):
    wid = lax.axis_index("s") * _NC + lax.axis_index("c")
    base = wid * _BPW
    pltpu.sync_copy(idx_hbm.at[pl.ds(wid * _NCHUNK, _NCHUNK)], idx_v)
    bufs = (buf0, buf1)
    gsem = (g0, g1)
    ssem = (s0, s1)
    gat = [None, None]
    sto = [None, None]
    gat[0] = pltpu.async_copy(table_hbm.at[idx_v.at[0]], buf0, g0)
    for j in range(_NCHUNK):
        cur = j & 1
        nxt = 1 - cur
        if j + 1 < _NCHUNK:
            if sto[nxt] is not None:
                sto[nxt].wait()
            gat[nxt] = pltpu.async_copy(
                table_hbm.at[idx_v.at[j + 1]], bufs[nxt], gsem[nxt])
        gat[cur].wait()
        sto[cur] = pltpu.async_copy(
            bufs[cur], out_hbm.at[pl.ds(base + j * _CH, _CH)], ssem[cur])
    sto[0].wait()
    sto[1].wait()


def kernel(ids, indicator_embs):
    ids_2d = ids.reshape(_NW * _NCHUNK, _CH).astype(jnp.int32)
    out = _sc_gather(ids_2d, indicator_embs)
    return out.reshape(ids.shape + (_HIDDEN,))


# speedup vs baseline: 2.6810x; 2.6810x over previous
import functools

import jax
import jax.numpy as jnp
from jax import lax
from jax.experimental import pallas as pl
from jax.experimental.pallas import tpu as pltpu
from jax.experimental.pallas import tpu_sc as plsc

_NROWS = 8
_HIDDEN = 1024
_B = 4 * 4096
_NC, _NS = 2, 16
_NW = _NC * _NS
_BPW = _B // _NW
_CH = 32
_NCHUNK = _BPW // _CH

_mesh = plsc.VectorSubcoreMesh(core_axis_name="c", subcore_axis_name="s")


@functools.partial(
    pl.kernel,
    mesh=_mesh,
    out_type=jax.ShapeDtypeStruct((_B, _HIDDEN), jnp.float32),
    scratch_types=[
        pltpu.VMEM((_NCHUNK, _CH), jnp.int32),
        pltpu.VMEM((_CH, _HIDDEN), jnp.float32),
        pltpu.VMEM((_CH, _HIDDEN), jnp.float32),
        pltpu.SemaphoreType.DMA,
        pltpu.SemaphoreType.DMA,
        pltpu.SemaphoreType.DMA,
        pltpu.SemaphoreType.DMA,
    ],
)
def _sc_gather(idx_hbm, table_hbm, out_hbm, idx_v, buf0, buf1, g0, g1, s0, s1):
    wid = lax.axis_index("s") * _NC + lax.axis_index("c")
    base = wid * _BPW
    pltpu.sync_copy(idx_hbm.at[pl.ds(wid * _NCHUNK, _NCHUNK)], idx_v)
    bufs = (buf0, buf1)
    gsem = (g0, g1)
    ssem = (s0, s1)
    gat = [None, None]
    sto = [None, None]
    gat[0] = pltpu.async_copy(table_hbm.at[idx_v.at[0]], buf0, g0)
    for j in range(_NCHUNK):
        cur = j & 1
        nxt = 1 - cur
        if j + 1 < _NCHUNK:
            if sto[nxt] is not None:
                sto[nxt].wait()
            gat[nxt] = pltpu.async_copy(
                table_hbm.at[idx_v.at[j + 1]], bufs[nxt], gsem[nxt])
        gat[cur].wait()
        sto[cur] = pltpu.async_copy(
            bufs[cur], out_hbm.at[pl.ds(base + j * _CH, _CH)], ssem[cur])
    sto[0].wait()
    sto[1].wait()


def kernel(ids, indicator_embs):
    table_rep = jnp.tile(indicator_embs, (_NW, 1))
    ids_2d = ids.reshape(_NW * _NCHUNK, _CH).astype(jnp.int32)
    bias = _NROWS * (jnp.arange(_NW * _NCHUNK, dtype=jnp.int32) // _NCHUNK)
    ids_2d = ids_2d + bias[:, None]
    out = _sc_gather(ids_2d, table_rep)
    return out.reshape(ids.shape + (_HIDDEN,))
